# baseline (speedup 1.0000x reference)
import functools

import jax
import jax.numpy as jnp
from jax import lax
from jax.experimental import pallas as pl
from jax.experimental.pallas import tpu as pltpu
from jax.experimental.pallas import tpu_sc as plsc

_NC = 2
_NS = 16
_NW = _NC * _NS
_L = 16

_CW = 128
_NBUF = 3
_EPS = 1e-5


def _make_gather(n_chunks, d_model):
    per_w = n_chunks // _NW

    mesh = plsc.VectorSubcoreMesh(core_axis_name="c", subcore_axis_name="s")

    @functools.partial(
        pl.kernel,
        out_type=jax.ShapeDtypeStruct((_NW, per_w, _CW, d_model),
                                      jnp.float32),
        mesh=mesh,
        scratch_types=[
            pltpu.VMEM((per_w, _CW), jnp.int32),
            pltpu.VMEM((_CW, d_model), jnp.float32),
            pltpu.VMEM((_CW, d_model), jnp.float32),
            pltpu.VMEM((_CW, d_model), jnp.float32),
            pltpu.SemaphoreType.DMA,
            pltpu.SemaphoreType.DMA,
            pltpu.SemaphoreType.DMA,
            pltpu.SemaphoreType.DMA,
            pltpu.SemaphoreType.DMA,
            pltpu.SemaphoreType.DMA,
        ],
    )
    def gather_kernel(x_hbm, tok_hbm, out_hbm, idx_v,
                      rows0, rows1, rows2, gs0, gs1, gs2, os0, os1, os2):
        rows = (rows0, rows1, rows2)
        gsem = (gs0, gs1, gs2)
        osem = (os0, os1, os2)
        wid = lax.axis_index("s") * _NC + lax.axis_index("c")

        pltpu.sync_copy(x_hbm.at[wid], idx_v)

        def start_gather(c, b):
            pltpu.async_copy(tok_hbm.at[idx_v.at[c]], rows[b], gsem[b])

        def wait_gather(c, b):
            pltpu.make_async_copy(
                tok_hbm.at[idx_v.at[c]], rows[b], gsem[b]).wait()

        def start_out(c, b):
            pltpu.async_copy(rows[b], out_hbm.at[wid, c], osem[b])

        def wait_out(c, b):
            pltpu.make_async_copy(
                rows[b], out_hbm.at[wid, c], osem[b]).wait()

        start_gather(0, 0)
        start_gather(1, 1)

        def slot(c, b, with_gather):
            wait_gather(c, b)
            start_out(c, b)
            if with_gather:
                nb = (b + 2) % _NBUF

                @pl.when(c >= 1)
                def _():
                    wait_out(c - 1, nb)

                start_gather(c + 2, nb)

        def pipe_body(t, _):
            c = t * _NBUF
            for r in range(_NBUF):
                slot(c + r, r, True)
            return ()

        n_full = (per_w - 2) // _NBUF
        lax.fori_loop(0, n_full, pipe_body, (), unroll=False)
        for c in range(_NBUF * n_full, per_w):
            slot(c, c % _NBUF, c <= per_w - 3)
        for c in (per_w - 3, per_w - 2, per_w - 1):
            wait_out(c, c % _NBUF)

    return gather_kernel


_RBC = 8


def _make_ln(nc_total, nc, period, d_model, base, aliased):
    grid = (nc // _RBC,)
    bb = base // _RBC
    rows = _RBC * _CW

    def body(*refs):
        if aliased:
            refs = refs[1:]
        tok_ref, seg_ref, ca_ref, cd_ref, g_ref, b_ref, w_ref, out_ref = refs
        i = pl.program_id(0)
        p = lax.rem((bb + i) * _RBC, period)
        t3 = tok_ref[...]
        s3 = seg_ref[...][..., None]
        h3 = t3 + ca_ref[pl.ds(p, _RBC)] + s3 * cd_ref[pl.ds(p, _RBC)]
        h = h3.reshape(rows, d_model)
        w = w_ref[...]
        mean = jnp.dot(h, w, preferred_element_type=jnp.float32)
        msq = jnp.dot(h * h, w, preferred_element_type=jnp.float32)
        inv = lax.rsqrt(msq - mean * mean + _EPS)
        out_ref[...] = ((h - mean) * inv * g_ref[...][0][None]
                        + b_ref[...][0][None])

    ntbl = period + _RBC - 1
    in_specs = [
        pl.BlockSpec((_RBC, _CW, d_model), lambda i: (i, 0, 0)),
        pl.BlockSpec((_RBC, _CW), lambda i: (i, 0)),
        pl.BlockSpec((ntbl, _CW, d_model), lambda i: (0, 0, 0)),
        pl.BlockSpec((ntbl, _CW, d_model), lambda i: (0, 0, 0)),
        pl.BlockSpec((8, d_model), lambda i: (0, 0)),
        pl.BlockSpec((8, d_model), lambda i: (0, 0)),
        pl.BlockSpec((d_model, d_model), lambda i: (0, 0)),
    ]
    if aliased:
        in_specs = [pl.BlockSpec(memory_space=pl.ANY)] + in_specs
    return pl.pallas_call(
        body,
        grid=grid,
        in_specs=in_specs,
        out_specs=pl.BlockSpec((rows, d_model), lambda i: (i + bb, 0)),
        out_shape=jax.ShapeDtypeStruct((nc_total * _CW, d_model),
                                       jnp.float32),
        input_output_aliases={0: 0} if aliased else {},
    )


def kernel(x, seg, tok_table, pos_table, seg_table, gamma, beta):
    b, s = x.shape
    v, d = tok_table.shape

    n_chunks = b * s // _CW
    import math
    period = math.lcm(_CW, s) // _CW
    ntbl = period + _RBC - 1
    comb_a = pos_table + seg_table[0][None]
    comb_d = jnp.broadcast_to(seg_table[1] - seg_table[0], (s, d))
    pidx = (jnp.arange(ntbl)[:, None] * _CW + jnp.arange(_CW)[None, :]) % s
    ca_big = comb_a[pidx]
    cd_big = comb_d[pidx]
    g8 = jnp.broadcast_to(gamma, (8, d))
    b8 = jnp.broadcast_to(beta, (8, d))
    wmean = jnp.full((d, d), 1.0 / d, jnp.float32)

    xf = x.reshape(n_chunks, _CW).astype(jnp.int32)
    sf = seg.reshape(n_chunks, _CW).astype(jnp.float32)

    slices = (320, 640, 640)
    assert sum(slices) == n_chunks

    rows = []
    base = 0
    for nc in slices:
        xi = lax.slice_in_dim(xf, base, base + nc, axis=0)
        xi = xi.reshape(_NW, nc // _NW, _CW)
        rows.append(_make_gather(nc, d)(xi, tok_table).reshape(nc, _CW, d))
        base += nc
    out = None
    base = 0
    for i, nc in enumerate(slices):
        si = lax.slice_in_dim(sf, base, base + nc, axis=0)
        ln = _make_ln(n_chunks, nc, period, d, base, aliased=i > 0)
        args = (rows[i], si, ca_big, cd_big, g8, b8, wmean)
        out = ln(*args) if i == 0 else ln(out, *args)
        base += nc
    return out.reshape(b, s, d)


# speedup vs baseline: 1.4425x; 1.4425x over previous
import functools

import jax
import jax.numpy as jnp
from jax import lax
from jax.experimental import pallas as pl
from jax.experimental.pallas import tpu as pltpu
from jax.experimental.pallas import tpu_sc as plsc

_NC = 2
_NS = 16
_NW = _NC * _NS
_L = 16

_CW = 128
_NBUF = 4
_EPS = 1e-5
_NSLICE = 2
_RB = 16


def _make_gather(n_chunks, d_model):
    per_w = n_chunks // _NW

    mesh = plsc.VectorSubcoreMesh(core_axis_name="c", subcore_axis_name="s")

    @functools.partial(
        pl.kernel,
        out_type=jax.ShapeDtypeStruct((_NW, per_w, _CW, d_model),
                                      jnp.float32),
        mesh=mesh,
        scratch_types=[
            pltpu.VMEM((per_w, _CW), jnp.int32),
            pltpu.VMEM((_CW, d_model), jnp.float32),
            pltpu.VMEM((_CW, d_model), jnp.float32),
            pltpu.VMEM((_CW, d_model), jnp.float32),
            pltpu.VMEM((_CW, d_model), jnp.float32),
            pltpu.SemaphoreType.DMA,
            pltpu.SemaphoreType.DMA,
            pltpu.SemaphoreType.DMA,
            pltpu.SemaphoreType.DMA,
            pltpu.SemaphoreType.DMA,
            pltpu.SemaphoreType.DMA,
            pltpu.SemaphoreType.DMA,
            pltpu.SemaphoreType.DMA,
        ],
    )
    def gather_kernel(x_hbm, tok_hbm, out_hbm, idx_v,
                      rows0, rows1, rows2, rows3,
                      gs0, gs1, gs2, gs3, os0, os1, os2, os3):
        rows = (rows0, rows1, rows2, rows3)
        gsem = (gs0, gs1, gs2, gs3)
        osem = (os0, os1, os2, os3)
        wid = lax.axis_index("s") * _NC + lax.axis_index("c")

        pltpu.sync_copy(x_hbm.at[wid], idx_v)

        def start_gather(c, b):
            pltpu.async_copy(tok_hbm.at[idx_v.at[c]], rows[b], gsem[b])

        def wait_gather(c, b):
            pltpu.make_async_copy(
                tok_hbm.at[idx_v.at[c]], rows[b], gsem[b]).wait()

        def start_out(c, b):
            pltpu.async_copy(rows[b], out_hbm.at[wid, c], osem[b])

        def wait_out(c, b):
            pltpu.make_async_copy(
                rows[b], out_hbm.at[wid, c], osem[b]).wait()

        start_gather(0, 0)
        start_gather(1, 1)

        def slot(c, b, with_gather):
            wait_gather(c, b)
            start_out(c, b)
            if with_gather:
                nb = (b + 2) % _NBUF

                @pl.when(c >= 2)
                def _():
                    wait_out(c - 2, nb)

                start_gather(c + 2, nb)

        def pipe_body(t, _):
            c = t * _NBUF
            for r in range(_NBUF):
                slot(c + r, r, True)
            return ()

        n_full = (per_w - 2) // _NBUF
        lax.fori_loop(0, n_full, pipe_body, (), unroll=False)
        for c in range(_NBUF * n_full, per_w):
            slot(c, c % _NBUF, c <= per_w - 3)
        for c in range(per_w - _NBUF, per_w):
            wait_out(c, c % _NBUF)

    return gather_kernel


def _ln_block(tok_ref, seg_ref, comb_a_ref, comb_d_ref, g_ref, b_ref,
              w_ref, out_ref):
    t = tok_ref[...]
    s = seg_ref[...][..., None]
    h = t + comb_a_ref[...][None] + s * comb_d_ref[...][None]
    d = t.shape[-1]
    h2 = h.reshape(-1, d)
    w = w_ref[...]
    mean = jnp.dot(h2, w, preferred_element_type=jnp.float32)
    msq = jnp.dot(h2 * h2, w, preferred_element_type=jnp.float32)
    inv = lax.rsqrt(msq - mean * mean + _EPS)
    norm = (h2 - mean) * inv
    out2 = norm * g_ref[...][0][None] + b_ref[...][0][None]
    out_ref[...] = out2.reshape(t.shape)


def _make_ln(nb_total, nb, seq, d_model, base, aliased):
    grid = (nb // _RB,)
    bb = base // _RB

    def body(*refs):
        _ln_block(*refs[1:]) if aliased else _ln_block(*refs)

    in_specs = [
        pl.BlockSpec((_RB, seq, d_model), lambda i: (i, 0, 0)),
        pl.BlockSpec((_RB, seq), lambda i: (i, 0)),
        pl.BlockSpec((seq, d_model), lambda i: (0, 0)),
        pl.BlockSpec((seq, d_model), lambda i: (0, 0)),
        pl.BlockSpec((8, d_model), lambda i: (0, 0)),
        pl.BlockSpec((8, d_model), lambda i: (0, 0)),
        pl.BlockSpec((d_model, d_model), lambda i: (0, 0)),
    ]
    if aliased:
        in_specs = [pl.BlockSpec(memory_space=pl.ANY)] + in_specs
    return pl.pallas_call(
        body,
        grid=grid,
        in_specs=in_specs,
        out_specs=pl.BlockSpec((_RB, seq, d_model), lambda i: (i + bb, 0, 0)),
        out_shape=jax.ShapeDtypeStruct((nb_total, seq, d_model),
                                       jnp.float32),
        input_output_aliases={0: 0} if aliased else {},
    )


def kernel(x, seg, tok_table, pos_table, seg_table, gamma, beta):
    b, s = x.shape
    v, d = tok_table.shape

    comb_a = pos_table + seg_table[0][None]
    comb_d = jnp.broadcast_to(seg_table[1] - seg_table[0], (s, d))
    g8 = jnp.broadcast_to(gamma, (8, d))
    b8 = jnp.broadcast_to(beta, (8, d))
    wmean = jnp.full((d, d), 1.0 / d, jnp.float32)

    bs = b // _NSLICE
    n_chunks = bs * s // _CW
    per_w = n_chunks // _NW
    gather = _make_gather(n_chunks, d)

    rows = []
    for i in range(_NSLICE):
        xi = lax.slice_in_dim(x, i * bs, (i + 1) * bs, axis=0)
        xi = xi.reshape(_NW, per_w, _CW).astype(jnp.int32)
        rows.append(gather(xi, tok_table).reshape(bs, s, d))
    out = None
    for i in range(_NSLICE):
        si = lax.slice_in_dim(seg, i * bs, (i + 1) * bs, axis=0)
        ln = _make_ln(b, bs, s, d, i * bs, aliased=i > 0)
        args = (rows[i], si.astype(jnp.float32), comb_a, comb_d,
                g8, b8, wmean)
        out = ln(*args) if i == 0 else ln(out, *args)
    return out


# speedup vs baseline: 1.4441x; 1.0011x over previous
import functools

import jax
import jax.numpy as jnp
from jax import lax
from jax.experimental import pallas as pl
from jax.experimental.pallas import tpu as pltpu
from jax.experimental.pallas import tpu_sc as plsc

_NC = 2
_NS = 16
_NW = _NC * _NS

_CW = 128
_NBUF = 4
_EPS = 1e-5
_NSLICE = 2
_RB = 16


def _make_gather(n_chunks, d_model):
    per_w = n_chunks // _NW

    mesh = plsc.VectorSubcoreMesh(core_axis_name="c", subcore_axis_name="s")

    @functools.partial(
        pl.kernel,
        out_type=jax.ShapeDtypeStruct((_NW, per_w, _CW, d_model),
                                      jnp.float32),
        mesh=mesh,
        scratch_types=[
            pltpu.VMEM((per_w, _CW), jnp.int32),
            pltpu.VMEM((_CW, d_model), jnp.float32),
            pltpu.VMEM((_CW, d_model), jnp.float32),
            pltpu.VMEM((_CW, d_model), jnp.float32),
            pltpu.VMEM((_CW, d_model), jnp.float32),
            pltpu.SemaphoreType.DMA,
            pltpu.SemaphoreType.DMA,
            pltpu.SemaphoreType.DMA,
            pltpu.SemaphoreType.DMA,
            pltpu.SemaphoreType.DMA,
            pltpu.SemaphoreType.DMA,
            pltpu.SemaphoreType.DMA,
            pltpu.SemaphoreType.DMA,
        ],
    )
    def gather_kernel(x_hbm, tok_hbm, out_hbm, idx_v,
                      rows0, rows1, rows2, rows3,
                      gs0, gs1, gs2, gs3, os0, os1, os2, os3):
        rows = (rows0, rows1, rows2, rows3)
        gsem = (gs0, gs1, gs2, gs3)
        osem = (os0, os1, os2, os3)
        wid = lax.axis_index("s") * _NC + lax.axis_index("c")

        pltpu.sync_copy(x_hbm.at[wid], idx_v)

        def start_gather(c, b):
            pltpu.async_copy(tok_hbm.at[idx_v.at[c]], rows[b], gsem[b])

        def wait_gather(c, b):
            pltpu.make_async_copy(
                tok_hbm.at[idx_v.at[c]], rows[b], gsem[b]).wait()

        def start_out(c, b):
            pltpu.async_copy(rows[b], out_hbm.at[wid, c], osem[b])

        def wait_out(c, b):
            pltpu.make_async_copy(
                rows[b], out_hbm.at[wid, c], osem[b]).wait()

        start_gather(0, 0)
        start_gather(1, 1)

        def slot(c, b, with_gather):
            wait_gather(c, b)
            start_out(c, b)
            if with_gather:
                nb = (b + 2) % _NBUF

                @pl.when(c >= 2)
                def _():
                    wait_out(c - 2, nb)

                start_gather(c + 2, nb)

        def pipe_body(t, _):
            c = t * _NBUF
            for r in range(_NBUF):
                slot(c + r, r, True)
            return ()

        n_full = (per_w - 2) // _NBUF
        lax.fori_loop(0, n_full, pipe_body, (), unroll=False)
        for c in range(_NBUF * n_full, per_w):
            slot(c, c % _NBUF, c <= per_w - 3)
        for c in range(per_w - _NBUF, per_w):
            wait_out(c, c % _NBUF)

    return gather_kernel


def _ln_block(tok_ref, seg_ref, comb_a_ref, comb_d_ref, g_ref, b_ref,
              w_ref, out_ref):
    t = tok_ref[...]
    s = seg_ref[...][..., None]
    h = t + comb_a_ref[...][None] + s * comb_d_ref[...][None]
    d = t.shape[-1]
    h2 = h.reshape(-1, d)
    w = w_ref[...]
    mean = jnp.dot(h2, w, preferred_element_type=jnp.float32)
    msq = jnp.dot(h2 * h2, w, preferred_element_type=jnp.float32)
    inv = lax.rsqrt(msq - mean * mean + _EPS)
    norm = (h2 - mean) * inv
    out2 = norm * g_ref[...][0][None] + b_ref[...][0][None]
    out_ref[...] = out2.reshape(t.shape)


def _make_ln(nb_total, nb, seq, d_model, base, aliased):
    grid = (nb // _RB,)
    bb = base // _RB

    def body(*refs):
        _ln_block(*refs[1:]) if aliased else _ln_block(*refs)

    in_specs = [
        pl.BlockSpec((_RB, seq, d_model), lambda i: (i, 0, 0)),
        pl.BlockSpec((_RB, seq), lambda i: (i, 0)),
        pl.BlockSpec((seq, d_model), lambda i: (0, 0)),
        pl.BlockSpec((seq, d_model), lambda i: (0, 0)),
        pl.BlockSpec((8, d_model), lambda i: (0, 0)),
        pl.BlockSpec((8, d_model), lambda i: (0, 0)),
        pl.BlockSpec((d_model, d_model), lambda i: (0, 0)),
    ]
    if aliased:
        in_specs = [pl.BlockSpec(memory_space=pl.ANY)] + in_specs
    return pl.pallas_call(
        body,
        grid=grid,
        in_specs=in_specs,
        out_specs=pl.BlockSpec((_RB, seq, d_model), lambda i: (i + bb, 0, 0)),
        out_shape=jax.ShapeDtypeStruct((nb_total, seq, d_model),
                                       jnp.float32),
        input_output_aliases={0: 0} if aliased else {},
    )


def kernel(x, seg, tok_table, pos_table, seg_table, gamma, beta):
    b, s = x.shape
    v, d = tok_table.shape

    comb_a = pos_table + seg_table[0][None]
    comb_d = jnp.broadcast_to(seg_table[1] - seg_table[0], (s, d))
    g8 = jnp.broadcast_to(gamma, (8, d))
    b8 = jnp.broadcast_to(beta, (8, d))
    wmean = jnp.full((d, d), 1.0 / d, jnp.float32)

    bs = b // _NSLICE
    n_chunks = bs * s // _CW
    per_w = n_chunks // _NW
    gather = _make_gather(n_chunks, d)

    rows = []
    for i in range(_NSLICE):
        xi = lax.slice_in_dim(x, i * bs, (i + 1) * bs, axis=0)
        xi = xi.reshape(_NW, per_w, _CW).astype(jnp.int32)
        rows.append(gather(xi, tok_table).reshape(bs, s, d))
    out = None
    for i in range(_NSLICE):
        si = lax.slice_in_dim(seg, i * bs, (i + 1) * bs, axis=0)
        ln = _make_ln(b, bs, s, d, i * bs, aliased=i > 0)
        args = (rows[i], si.astype(jnp.float32), comb_a, comb_d,
                g8, b8, wmean)
        out = ln(*args) if i == 0 else ln(out, *args)
    return out


# speedup vs baseline: 1.5644x; 1.0833x over previous
import functools

import jax
import jax.numpy as jnp
from jax import lax
from jax.experimental import pallas as pl
from jax.experimental.pallas import tpu as pltpu
from jax.experimental.pallas import tpu_sc as plsc

_NC = 2
_NS = 16
_NW = _NC * _NS

_CW = 128
_NBUF = 4
_EPS = 1e-5
_NSLICE = 2
_RB = 32


def _make_gather(n_chunks, d_model):
    per_w = n_chunks // _NW

    mesh = plsc.VectorSubcoreMesh(core_axis_name="c", subcore_axis_name="s")

    @functools.partial(
        pl.kernel,
        out_type=jax.ShapeDtypeStruct((_NW, per_w, _CW, d_model),
                                      jnp.float32),
        mesh=mesh,
        scratch_types=[
            pltpu.VMEM((per_w, _CW), jnp.int32),
            pltpu.VMEM((_CW, d_model), jnp.float32),
            pltpu.VMEM((_CW, d_model), jnp.float32),
            pltpu.VMEM((_CW, d_model), jnp.float32),
            pltpu.VMEM((_CW, d_model), jnp.float32),
            pltpu.SemaphoreType.DMA,
            pltpu.SemaphoreType.DMA,
            pltpu.SemaphoreType.DMA,
            pltpu.SemaphoreType.DMA,
            pltpu.SemaphoreType.DMA,
            pltpu.SemaphoreType.DMA,
            pltpu.SemaphoreType.DMA,
            pltpu.SemaphoreType.DMA,
        ],
    )
    def gather_kernel(x_hbm, tok_hbm, out_hbm, idx_v,
                      rows0, rows1, rows2, rows3,
                      gs0, gs1, gs2, gs3, os0, os1, os2, os3):
        rows = (rows0, rows1, rows2, rows3)
        gsem = (gs0, gs1, gs2, gs3)
        osem = (os0, os1, os2, os3)
        wid = lax.axis_index("s") * _NC + lax.axis_index("c")

        pltpu.sync_copy(x_hbm.at[wid], idx_v)

        def start_gather(c, b):
            pltpu.async_copy(tok_hbm.at[idx_v.at[c]], rows[b], gsem[b])

        def wait_gather(c, b):
            pltpu.make_async_copy(
                tok_hbm.at[idx_v.at[c]], rows[b], gsem[b]).wait()

        def start_out(c, b):
            pltpu.async_copy(rows[b], out_hbm.at[wid, c], osem[b])

        def wait_out(c, b):
            pltpu.make_async_copy(
                rows[b], out_hbm.at[wid, c], osem[b]).wait()

        start_gather(0, 0)
        start_gather(1, 1)

        def slot(c, b, with_gather):
            wait_gather(c, b)
            start_out(c, b)
            if with_gather:
                nb = (b + 2) % _NBUF

                @pl.when(c >= 2)
                def _():
                    wait_out(c - 2, nb)

                start_gather(c + 2, nb)

        def pipe_body(t, _):
            c = t * _NBUF
            for r in range(_NBUF):
                slot(c + r, r, True)
            return ()

        n_full = (per_w - 2) // _NBUF
        lax.fori_loop(0, n_full, pipe_body, (), unroll=False)
        for c in range(_NBUF * n_full, per_w):
            slot(c, c % _NBUF, c <= per_w - 3)
        for c in range(per_w - _NBUF, per_w):
            wait_out(c, c % _NBUF)

    return gather_kernel


def _ln_block(tok_ref, seg_ref, comb_a_ref, comb_d_ref, g_ref, b_ref,
              w_ref, out_ref):
    t = tok_ref[...]
    s = seg_ref[...][..., None]
    h = t + comb_a_ref[...][None] + s * comb_d_ref[...][None]
    d = t.shape[-1]
    h2 = h.reshape(-1, d)
    w = w_ref[...]
    mean = jnp.dot(h2, w, preferred_element_type=jnp.float32)
    msq = jnp.dot(h2 * h2, w, preferred_element_type=jnp.float32)
    inv = lax.rsqrt(msq - mean * mean + _EPS)
    norm = (h2 - mean) * inv
    out2 = norm * g_ref[...][0][None] + b_ref[...][0][None]
    out_ref[...] = out2.reshape(t.shape)


def _make_ln(nb_total, nb, seq, d_model, base, aliased):
    grid = (nb // _RB,)
    bb = base // _RB

    def body(*refs):
        _ln_block(*refs[1:]) if aliased else _ln_block(*refs)

    in_specs = [
        pl.BlockSpec((_RB, seq, d_model), lambda i: (i, 0, 0)),
        pl.BlockSpec((_RB, seq), lambda i: (i, 0)),
        pl.BlockSpec((seq, d_model), lambda i: (0, 0)),
        pl.BlockSpec((seq, d_model), lambda i: (0, 0)),
        pl.BlockSpec((8, d_model), lambda i: (0, 0)),
        pl.BlockSpec((8, d_model), lambda i: (0, 0)),
        pl.BlockSpec((d_model, d_model), lambda i: (0, 0)),
    ]
    if aliased:
        in_specs = [pl.BlockSpec(memory_space=pl.ANY)] + in_specs
    return pl.pallas_call(
        body,
        grid=grid,
        in_specs=in_specs,
        out_specs=pl.BlockSpec((_RB, seq, d_model), lambda i: (i + bb, 0, 0)),
        out_shape=jax.ShapeDtypeStruct((nb_total, seq, d_model),
                                       jnp.float32),
        input_output_aliases={0: 0} if aliased else {},
    )


def kernel(x, seg, tok_table, pos_table, seg_table, gamma, beta):
    b, s = x.shape
    v, d = tok_table.shape

    comb_a = pos_table + seg_table[0][None]
    comb_d = jnp.broadcast_to(seg_table[1] - seg_table[0], (s, d))
    g8 = jnp.broadcast_to(gamma, (8, d))
    b8 = jnp.broadcast_to(beta, (8, d))
    wmean = jnp.full((d, d), 1.0 / d, jnp.float32)

    bs = b // _NSLICE
    n_chunks = bs * s // _CW
    per_w = n_chunks // _NW
    gather = _make_gather(n_chunks, d)

    rows = []
    for i in range(_NSLICE):
        xi = lax.slice_in_dim(x, i * bs, (i + 1) * bs, axis=0)
        xi = xi.reshape(_NW, per_w, _CW).astype(jnp.int32)
        rows.append(gather(xi, tok_table).reshape(bs, s, d))
    out = None
    for i in range(_NSLICE):
        si = lax.slice_in_dim(seg, i * bs, (i + 1) * bs, axis=0)
        ln = _make_ln(b, bs, s, d, i * bs, aliased=i > 0)
        args = (rows[i], si.astype(jnp.float32), comb_a, comb_d,
                g8, b8, wmean)
        out = ln(*args) if i == 0 else ln(out, *args)
    return out


# speedup vs baseline: 1.6174x; 1.0339x over previous
import functools

import jax
import jax.numpy as jnp
from jax import lax
from jax.experimental import pallas as pl
from jax.experimental.pallas import tpu as pltpu
from jax.experimental.pallas import tpu_sc as plsc

_NC = 2
_NS = 16
_NW = _NC * _NS

_CW = 128
_NBUF = 4
_EPS = 1e-5
_NSLICE = 2
_RB = 64


def _make_gather(n_chunks, d_model):
    per_w = n_chunks // _NW

    mesh = plsc.VectorSubcoreMesh(core_axis_name="c", subcore_axis_name="s")

    @functools.partial(
        pl.kernel,
        out_type=jax.ShapeDtypeStruct((_NW, per_w, _CW, d_model),
                                      jnp.float32),
        mesh=mesh,
        scratch_types=[
            pltpu.VMEM((per_w, _CW), jnp.int32),
            pltpu.VMEM((_CW, d_model), jnp.float32),
            pltpu.VMEM((_CW, d_model), jnp.float32),
            pltpu.VMEM((_CW, d_model), jnp.float32),
            pltpu.VMEM((_CW, d_model), jnp.float32),
            pltpu.SemaphoreType.DMA,
            pltpu.SemaphoreType.DMA,
            pltpu.SemaphoreType.DMA,
            pltpu.SemaphoreType.DMA,
            pltpu.SemaphoreType.DMA,
            pltpu.SemaphoreType.DMA,
            pltpu.SemaphoreType.DMA,
            pltpu.SemaphoreType.DMA,
        ],
    )
    def gather_kernel(x_hbm, tok_hbm, out_hbm, idx_v,
                      rows0, rows1, rows2, rows3,
                      gs0, gs1, gs2, gs3, os0, os1, os2, os3):
        rows = (rows0, rows1, rows2, rows3)
        gsem = (gs0, gs1, gs2, gs3)
        osem = (os0, os1, os2, os3)
        wid = lax.axis_index("s") * _NC + lax.axis_index("c")

        pltpu.sync_copy(x_hbm.at[wid], idx_v)

        def start_gather(c, b):
            pltpu.async_copy(tok_hbm.at[idx_v.at[c]], rows[b], gsem[b])

        def wait_gather(c, b):
            pltpu.make_async_copy(
                tok_hbm.at[idx_v.at[c]], rows[b], gsem[b]).wait()

        def start_out(c, b):
            pltpu.async_copy(rows[b], out_hbm.at[wid, c], osem[b])

        def wait_out(c, b):
            pltpu.make_async_copy(
                rows[b], out_hbm.at[wid, c], osem[b]).wait()

        start_gather(0, 0)
        start_gather(1, 1)

        def slot(c, b, with_gather):
            wait_gather(c, b)
            start_out(c, b)
            if with_gather:
                nb = (b + 2) % _NBUF

                @pl.when(c >= 2)
                def _():
                    wait_out(c - 2, nb)

                start_gather(c + 2, nb)

        def pipe_body(t, _):
            c = t * _NBUF
            for r in range(_NBUF):
                slot(c + r, r, True)
            return ()

        n_full = (per_w - 2) // _NBUF
        lax.fori_loop(0, n_full, pipe_body, (), unroll=False)
        for c in range(_NBUF * n_full, per_w):
            slot(c, c % _NBUF, c <= per_w - 3)
        for c in range(per_w - _NBUF, per_w):
            wait_out(c, c % _NBUF)

    return gather_kernel


def _ln_block(tok_ref, seg_ref, comb_a_ref, comb_d_ref, g_ref, b_ref,
              w_ref, out_ref):
    t = tok_ref[...]
    s = seg_ref[...][..., None]
    h = t + comb_a_ref[...][None] + s * comb_d_ref[...][None]
    d = t.shape[-1]
    h2 = h.reshape(-1, d)
    w = w_ref[...]
    mean = jnp.dot(h2, w, preferred_element_type=jnp.float32)
    msq = jnp.dot(h2 * h2, w, preferred_element_type=jnp.float32)
    inv = lax.rsqrt(msq - mean * mean + _EPS)
    norm = (h2 - mean) * inv
    out2 = norm * g_ref[...][0][None] + b_ref[...][0][None]
    out_ref[...] = out2.reshape(t.shape)


def _make_ln(nb_total, nb, seq, d_model, base, aliased):
    grid = (nb // _RB,)
    bb = base // _RB

    def body(*refs):
        _ln_block(*refs[1:]) if aliased else _ln_block(*refs)

    in_specs = [
        pl.BlockSpec((_RB, seq, d_model), lambda i: (i, 0, 0)),
        pl.BlockSpec((_RB, seq), lambda i: (i, 0)),
        pl.BlockSpec((seq, d_model), lambda i: (0, 0)),
        pl.BlockSpec((seq, d_model), lambda i: (0, 0)),
        pl.BlockSpec((8, d_model), lambda i: (0, 0)),
        pl.BlockSpec((8, d_model), lambda i: (0, 0)),
        pl.BlockSpec((d_model, d_model), lambda i: (0, 0)),
    ]
    if aliased:
        in_specs = [pl.BlockSpec(memory_space=pl.ANY)] + in_specs
    return pl.pallas_call(
        body,
        grid=grid,
        in_specs=in_specs,
        out_specs=pl.BlockSpec((_RB, seq, d_model), lambda i: (i + bb, 0, 0)),
        out_shape=jax.ShapeDtypeStruct((nb_total, seq, d_model),
                                       jnp.float32),
        input_output_aliases={0: 0} if aliased else {},
    )


def kernel(x, seg, tok_table, pos_table, seg_table, gamma, beta):
    b, s = x.shape
    v, d = tok_table.shape

    comb_a = pos_table + seg_table[0][None]
    comb_d = jnp.broadcast_to(seg_table[1] - seg_table[0], (s, d))
    g8 = jnp.broadcast_to(gamma, (8, d))
    b8 = jnp.broadcast_to(beta, (8, d))
    wmean = jnp.full((d, d), 1.0 / d, jnp.float32)

    bs = b // _NSLICE
    n_chunks = bs * s // _CW
    per_w = n_chunks // _NW
    gather = _make_gather(n_chunks, d)

    rows = []
    for i in range(_NSLICE):
        xi = lax.slice_in_dim(x, i * bs, (i + 1) * bs, axis=0)
        xi = xi.reshape(_NW, per_w, _CW).astype(jnp.int32)
        rows.append(gather(xi, tok_table).reshape(bs, s, d))
    out = None
    for i in range(_NSLICE):
        si = lax.slice_in_dim(seg, i * bs, (i + 1) * bs, axis=0)
        ln = _make_ln(b, bs, s, d, i * bs, aliased=i > 0)
        args = (rows[i], si.astype(jnp.float32), comb_a, comb_d,
                g8, b8, wmean)
        out = ln(*args) if i == 0 else ln(out, *args)
    return out
